# baseline (speedup 1.0000x reference)
import jax
import jax.numpy as jnp
from jax.experimental import pallas as pl
from jax.experimental.pallas import tpu as pltpu


def kernel(loc_preds, loc_targets, cls_preds, cls_targets):
    B, A = cls_targets.shape
    C = cls_preds.shape[-1]
    L = (A * 4) // 8

    cls_t = jnp.transpose(cls_preds, (0, 2, 1))
    tgt = cls_targets.astype(jnp.int32)[:, None, :]

    lp8 = loc_preds.reshape(B * 8, L)
    lt8 = loc_targets.reshape(B * 8, L)
    pos4 = jnp.repeat((cls_targets > 0).astype(jnp.float32), 4,
                      axis=1).reshape(B * 8, L)

    def body(cls_ref, tgt_ref, lp_ref, lt_ref, m4_ref, oloc_ref, ocls_ref,
             ce_s, pos_s, loc_acc):
        b = pl.program_id(0)

        logits = cls_ref[0]
        ti = tgt_ref[0]
        pos_f = jnp.where(ti > 0, 1.0, 0.0)

        s = jnp.sum(jnp.exp(logits), axis=0, keepdims=True)
        cidx = jax.lax.broadcasted_iota(jnp.int32, (C, A), 0)
        tl = jnp.sum(jnp.where(cidx == ti, logits, 0.0),
                     axis=0, keepdims=True)
        ce = jnp.maximum(jnp.log(s) - tl, 0.0)

        ce_s[b, :] = ce[0]
        pos_s[b, :] = pos_f[0]

        d = lp_ref[...] - lt_ref[...]
        ad = jnp.abs(d)
        q = jnp.minimum(ad, 1.0)
        sl = 0.5 * q * q + jnp.maximum(ad - 1.0, 0.0)
        contrib = sl * m4_ref[...]

        @pl.when(b == 0)
        def _init():
            loc_acc[...] = jnp.zeros_like(loc_acc)

        loc_acc[...] = loc_acc[...] + contrib

        @pl.when(b == B - 1)
        def _phase2():
            cem = ce_s[...]
            posm = pos_s[...]
            npos = jnp.sum(posm, axis=1, keepdims=True)
            npt = jnp.sum(npos, keepdims=True)
            pos_sum = jnp.sum(cem * posm, keepdims=True)
            neg = cem * (1.0 - posm)
            ni = jax.lax.bitcast_convert_type(neg, jnp.int32)
            kf = jnp.clip(3.0 * npos, 1.0, float(A - 1))

            def bit_step(i, t):
                cand = t | (jnp.int32(1) << (30 - i))
                cnt = jnp.sum(jnp.where(ni >= cand, 1.0, 0.0),
                              axis=1, keepdims=True)
                return jnp.where(cnt >= kf, cand, t)

            v = jax.lax.fori_loop(0, 31, bit_step,
                                  jnp.zeros((B, 1), jnp.int32))
            vf = jax.lax.bitcast_convert_type(v, jnp.float32)
            gt = ni > v
            cnt_gt = jnp.sum(jnp.where(gt, 1.0, 0.0), axis=1, keepdims=True)
            sum_gt = jnp.sum(jnp.where(gt, neg, 0.0), axis=1, keepdims=True)
            neg_sum = jnp.sum(sum_gt + (kf - cnt_gt) * vf, keepdims=True)

            loc_total = jnp.sum(loc_acc[...], keepdims=True)
            oloc_ref[...] = 20.0 * loc_total / npt
            ocls_ref[...] = (pos_sum + neg_sum) / npt

    out_loc, out_cls = pl.pallas_call(
        body,
        grid=(B,),
        in_specs=[
            pl.BlockSpec((1, C, A), lambda b: (b, 0, 0)),
            pl.BlockSpec((1, 1, A), lambda b: (b, 0, 0)),
            pl.BlockSpec((8, L), lambda b: (b, 0)),
            pl.BlockSpec((8, L), lambda b: (b, 0)),
            pl.BlockSpec((8, L), lambda b: (b, 0)),
        ],
        out_specs=[
            pl.BlockSpec((1, 1), lambda b: (0, 0)),
            pl.BlockSpec((1, 1), lambda b: (0, 0)),
        ],
        out_shape=[
            jax.ShapeDtypeStruct((1, 1), jnp.float32),
            jax.ShapeDtypeStruct((1, 1), jnp.float32),
        ],
        scratch_shapes=[
            pltpu.VMEM((B, A), jnp.float32),
            pltpu.VMEM((B, A), jnp.float32),
            pltpu.VMEM((8, L), jnp.float32),
        ],
    )(cls_t, tgt, lp8, lt8, pos4)

    return (out_loc[0, 0], out_cls[0, 0])


# speedup vs baseline: 14.4817x; 14.4817x over previous
import jax
import jax.numpy as jnp
from jax.experimental import pallas as pl
from jax.experimental.pallas import tpu as pltpu


def kernel(loc_preds, loc_targets, cls_preds, cls_targets):
    B, A = cls_targets.shape
    C = cls_preds.shape[-1]
    cls_t = jnp.transpose(cls_preds, (0, 2, 1))
    tgt = cls_targets.astype(jnp.int32)[:, None, :]
    d_t = jnp.transpose(loc_preds - loc_targets, (0, 2, 1))

    def body(cls_ref, tgt_ref, d_ref, oloc_ref, ocls_ref,
             ce_s, pos_s, loc_acc):
        b = pl.program_id(0)

        logits = cls_ref[0]
        ti = tgt_ref[0]
        pos_f = jnp.where(ti > 0, 1.0, 0.0)

        s = jnp.sum(jnp.exp(logits), axis=0, keepdims=True)
        cidx = jax.lax.broadcasted_iota(jnp.int32, (C, A), 0)
        tl = jnp.sum(jnp.where(cidx == ti, logits, 0.0),
                     axis=0, keepdims=True)
        ce = jnp.maximum(jnp.log(s) - tl, 0.0)

        ce_s[b, :] = ce[0]
        pos_s[b, :] = pos_f[0]

        ad = jnp.abs(d_ref[0])
        q = jnp.minimum(ad, 1.0)
        sl = 0.5 * q * q + jnp.maximum(ad - 1.0, 0.0)
        contrib = jnp.where(pos_f > 0.0, sl, 0.0)

        @pl.when(b == 0)
        def _init():
            loc_acc[...] = jnp.zeros_like(loc_acc)

        loc_acc[...] = loc_acc[...] + contrib

        @pl.when(b == B - 1)
        def _phase2():
            cem = ce_s[...]
            posm = pos_s[...]
            npos = jnp.sum(posm, axis=1, keepdims=True)
            npt = jnp.sum(npos, keepdims=True)
            pos_sum = jnp.sum(cem * posm, keepdims=True)
            neg = cem * (1.0 - posm)
            ni = jax.lax.bitcast_convert_type(neg, jnp.int32)
            kf = jnp.clip(3.0 * npos, 1.0, float(A - 1))

            def bit_step(i, t):
                cand = t | (jnp.int32(1) << (30 - i))
                cnt = jnp.sum(jnp.where(ni >= cand, 1.0, 0.0),
                              axis=1, keepdims=True)
                return jnp.where(cnt >= kf, cand, t)

            v = jax.lax.fori_loop(0, 31, bit_step,
                                  jnp.zeros((B, 1), jnp.int32))
            vf = jax.lax.bitcast_convert_type(v, jnp.float32)
            gt = ni > v
            cnt_gt = jnp.sum(jnp.where(gt, 1.0, 0.0), axis=1, keepdims=True)
            sum_gt = jnp.sum(jnp.where(gt, neg, 0.0), axis=1, keepdims=True)
            neg_sum = jnp.sum(sum_gt + (kf - cnt_gt) * vf, keepdims=True)

            loc_total = jnp.sum(loc_acc[...], keepdims=True)
            oloc_ref[...] = 20.0 * loc_total / npt
            ocls_ref[...] = (pos_sum + neg_sum) / npt

    out_loc, out_cls = pl.pallas_call(
        body,
        grid=(B,),
        in_specs=[
            pl.BlockSpec((1, C, A), lambda b: (b, 0, 0)),
            pl.BlockSpec((1, 1, A), lambda b: (b, 0, 0)),
            pl.BlockSpec((1, 4, A), lambda b: (b, 0, 0)),
        ],
        out_specs=[
            pl.BlockSpec((1, 1), lambda b: (0, 0)),
            pl.BlockSpec((1, 1), lambda b: (0, 0)),
        ],
        out_shape=[
            jax.ShapeDtypeStruct((1, 1), jnp.float32),
            jax.ShapeDtypeStruct((1, 1), jnp.float32),
        ],
        scratch_shapes=[
            pltpu.VMEM((B, A), jnp.float32),
            pltpu.VMEM((B, A), jnp.float32),
            pltpu.VMEM((4, A), jnp.float32),
        ],
    )(cls_t, tgt, d_t)

    return (out_loc[0, 0], out_cls[0, 0])


# speedup vs baseline: 16.9981x; 1.1738x over previous
import jax
import jax.numpy as jnp
from jax.experimental import pallas as pl
from jax.experimental.pallas import tpu as pltpu


def kernel(loc_preds, loc_targets, cls_preds, cls_targets):
    B, A = cls_targets.shape
    C = cls_preds.shape[-1]
    cls_t = jnp.transpose(cls_preds, (0, 2, 1))
    tgt = cls_targets.astype(jnp.int32)[:, None, :]
    lp_t = jnp.transpose(loc_preds, (0, 2, 1))
    lt_t = jnp.transpose(loc_targets, (0, 2, 1))

    def body(cls_ref, tgt_ref, lp_ref, lt_ref, oloc_ref, ocls_ref,
             ce_s, pos_s, loc_acc):
        b = pl.program_id(0)

        logits = cls_ref[0]
        ti = tgt_ref[0]
        pos_f = jnp.where(ti > 0, 1.0, 0.0)

        s = jnp.sum(jnp.exp(logits), axis=0, keepdims=True)
        cidx = jax.lax.broadcasted_iota(jnp.int32, (C, A), 0)
        tl = jnp.sum(jnp.where(cidx == ti, logits, 0.0),
                     axis=0, keepdims=True)
        ce = jnp.maximum(jnp.log(s) - tl, 0.0)

        ce_s[b, :] = ce[0]
        pos_s[b, :] = pos_f[0]

        ad = jnp.abs(lp_ref[0] - lt_ref[0])
        q = jnp.minimum(ad, 1.0)
        sl = 0.5 * q * q + jnp.maximum(ad - 1.0, 0.0)
        contrib = jnp.where(pos_f > 0.0, sl, 0.0)

        @pl.when(b == 0)
        def _init():
            loc_acc[...] = jnp.zeros_like(loc_acc)

        loc_acc[...] = loc_acc[...] + contrib

        @pl.when(b == B - 1)
        def _phase2():
            cem = ce_s[...]
            posm = pos_s[...]
            npos = jnp.sum(posm, axis=1, keepdims=True)
            npt = jnp.sum(npos, keepdims=True)
            pos_sum = jnp.sum(cem * posm, keepdims=True)
            neg = cem * (1.0 - posm)
            ni = jax.lax.bitcast_convert_type(neg, jnp.int32)
            kf = jnp.clip(3.0 * npos, 1.0, float(A - 1))

            n_nz = jnp.sum(jnp.where(ni > 0, 1.0, 0.0),
                           axis=1, keepdims=True)
            need = jnp.sum(jnp.where(n_nz > kf, 1.0, 0.0), keepdims=True)

            def bit_step(i, t):
                cand = t | (jnp.int32(1) << (30 - i))
                cnt = jnp.sum(jnp.where(ni >= cand, 1.0, 0.0),
                              axis=1, keepdims=True)
                return jnp.where(cnt >= kf, cand, t)

            v = jax.lax.cond(
                need[0, 0] > 0.0,
                lambda: jax.lax.fori_loop(0, 31, bit_step,
                                          jnp.zeros((B, 1), jnp.int32)),
                lambda: jnp.zeros((B, 1), jnp.int32))
            vf = jax.lax.bitcast_convert_type(v, jnp.float32)
            gt = ni > v
            cnt_gt = jnp.sum(jnp.where(gt, 1.0, 0.0), axis=1, keepdims=True)
            sum_gt = jnp.sum(jnp.where(gt, neg, 0.0), axis=1, keepdims=True)
            neg_sum = jnp.sum(sum_gt + (kf - cnt_gt) * vf, keepdims=True)

            loc_total = jnp.sum(loc_acc[...], keepdims=True)
            oloc_ref[...] = 20.0 * loc_total / npt
            ocls_ref[...] = (pos_sum + neg_sum) / npt

    out_loc, out_cls = pl.pallas_call(
        body,
        grid=(B,),
        in_specs=[
            pl.BlockSpec((1, C, A), lambda b: (b, 0, 0)),
            pl.BlockSpec((1, 1, A), lambda b: (b, 0, 0)),
            pl.BlockSpec((1, 4, A), lambda b: (b, 0, 0)),
            pl.BlockSpec((1, 4, A), lambda b: (b, 0, 0)),
        ],
        out_specs=[
            pl.BlockSpec((1, 1), lambda b: (0, 0)),
            pl.BlockSpec((1, 1), lambda b: (0, 0)),
        ],
        out_shape=[
            jax.ShapeDtypeStruct((1, 1), jnp.float32),
            jax.ShapeDtypeStruct((1, 1), jnp.float32),
        ],
        scratch_shapes=[
            pltpu.VMEM((B, A), jnp.float32),
            pltpu.VMEM((B, A), jnp.float32),
            pltpu.VMEM((4, A), jnp.float32),
        ],
    )(cls_t, tgt, lp_t, lt_t)

    return (out_loc[0, 0], out_cls[0, 0])
